# Initial kernel scaffold; baseline (speedup 1.0000x reference)
import jax
import jax.numpy as jnp
from jax.experimental import pallas as pl


def kernel(h, u, state_pos, action_pos, a2s_edge_index, s2s_edge_index, a2s_dis, s2s_dis, params):
    raise NotImplementedError("write your pallas kernel here")


# speedup vs baseline: 5.1836x; 5.1836x over previous
import functools

import jax
import jax.numpy as jnp
from jax import lax
from jax.experimental import pallas as pl
from jax.experimental.pallas import tpu as pltpu
from jax.experimental.pallas import tpu_sc as plsc

NS_N = 10000
E_N = 160000
F32 = jnp.float32

NC = 2
NSUB = 16
NW = NC * NSUB
CH = 128
NCHUNKS = E_N // CH
NPT = 1000
NSTRIPE = NS_N // NPT


def _pad_rows(w, total=256):
    return jnp.pad(w, ((0, total - w.shape[0]), (0, 0)))


def _split_first_layer(p_logit, p_msg):
    wl, bl = p_logit[0]['w'], p_logit[0]['b']
    wm, bm = p_msg[0]['w'], p_msg[0]['b']

    def side(lo1, hi1, lo2, hi2):
        return jnp.concatenate([
            jnp.concatenate([wl[lo1:hi1], wl[lo2:hi2]], axis=0),
            jnp.concatenate([wm[lo1:hi1], wm[lo2:hi2]], axis=0),
        ], axis=1)

    w_src = side(0, 2, 5, 133)
    w_dst = side(2, 4, 133, 261)
    w_dis = jnp.concatenate([wl[4], wm[4]]).reshape(1, 128)
    b = jnp.concatenate([bl, bm])
    return w_src, w_dst, w_dis, b


def _k1_body(xs_ref, xa_ref, ws_ref, bs_ref, wa_ref, q_ref, r_ref, s_ref, p_ref):
    y = jnp.dot(xs_ref[...], ws_ref[...], preferred_element_type=F32) + bs_ref[...]
    q_ref[...] = y[:, 0:128]
    r_ref[...] = y[:, 128:256]
    s_ref[...] = y[:, 256:384]
    p_ref[...] = jnp.dot(xa_ref[...], wa_ref[...], preferred_element_type=F32)


def _node_proj(xs, xa, ws, bs, wa):
    bm = 1000
    grid = NS_N // bm
    out = jax.ShapeDtypeStruct((NS_N, 128), F32)
    return pl.pallas_call(
        _k1_body,
        grid=(grid,),
        in_specs=[
            pl.BlockSpec((bm, 256), lambda i: (i, 0)),
            pl.BlockSpec((bm, 256), lambda i: (i, 0)),
            pl.BlockSpec((256, 384), lambda i: (0, 0)),
            pl.BlockSpec((1, 384), lambda i: (0, 0)),
            pl.BlockSpec((256, 128), lambda i: (0, 0)),
        ],
        out_specs=[pl.BlockSpec((bm, 128), lambda i: (i, 0))] * 4,
        out_shape=[out, out, out, out],
    )(xs, xa, ws, bs, wa)


@functools.cache
def _sc_mesh():
    return plsc.VectorSubcoreMesh(core_axis_name="c", subcore_axis_name="s")


@functools.cache
def _make_k2():
    return pl.kernel(
        _k2_body,
        out_type=jax.ShapeDtypeStruct((E_N, 128), F32),
        mesh=_sc_mesh(),
        scratch_types=[
            pltpu.VMEM((CH,), jnp.int32),
            pltpu.VMEM((CH,), jnp.int32),
            pltpu.VMEM((CH, 128), F32),
            pltpu.VMEM((CH, 128), F32),
            pltpu.SemaphoreType.DMA,
            pltpu.SemaphoreType.DMA,
        ],
    )


def _k2_gather(src, dst, ptab, qtab):
    return _make_k2()(src, dst, ptab, qtab)


def _k2_body(src_hbm, dst_hbm, ptab, qtab, out_hbm,
             idx_s, idx_d, rows_s, rows_d, sem1, sem2):
    wid = lax.axis_index("s") * NC + lax.axis_index("c")
    nch = jnp.where(wid < NCHUNKS - (NCHUNKS // NW) * NW, NCHUNKS // NW + 1,
                    NCHUNKS // NW)

    def chunk(i, _):
        base = (wid + NW * i) * CH
        pltpu.sync_copy(src_hbm.at[pl.ds(base, CH)], idx_s)
        pltpu.sync_copy(dst_hbm.at[pl.ds(base, CH)], idx_d)
        c1 = pltpu.async_copy(ptab.at[idx_s], rows_s, sem1)
        c2 = pltpu.async_copy(qtab.at[idx_d], rows_d, sem2)
        c1.wait()
        c2.wait()

        def erow(e, _):
            for j in range(8):
                sl = pl.ds(j * 16, 16)
                rows_s[e, sl] = rows_s[e, sl] + rows_d[e, sl]
            return 0

        lax.fori_loop(0, CH, erow, 0)
        pltpu.sync_copy(rows_s, out_hbm.at[pl.ds(base, CH)])
        return 0

    lax.fori_loop(0, nch, chunk, 0)


def _k3_body(pre_ref, dis_ref, wd_ref, w2l, b2l, w3l, b3l, w2m, b2m, w3m, b3m,
             out_ref):
    pre = pre_ref[...] + dis_ref[...] * wd_ref[...]
    xl = jnp.tanh(pre[:, 0:64])
    xm = jnp.tanh(pre[:, 64:128])
    tl = jnp.tanh(jnp.dot(xl, w2l[...], preferred_element_type=F32) + b2l[...])
    logit = jnp.dot(tl, w3l[...], preferred_element_type=F32) + b3l[...]
    e = jnp.exp(logit)
    tm = jnp.tanh(jnp.dot(xm, w2m[...], preferred_element_type=F32) + b2m[...])
    msg = jnp.dot(tm, w3m[...], preferred_element_type=F32) + b3m[...]
    out_ref[0] = e
    out_ref[1] = e * msg


def _edge_mlp(pre, dis, wd, p_logit, p_msg):
    bm = 640
    grid = E_N // bm
    w2l, b2l = p_logit[1]['w'], p_logit[1]['b'].reshape(1, 64)
    w3l, b3l = p_logit[2]['w'], p_logit[2]['b'].reshape(1, 128)
    w2m, b2m = p_msg[1]['w'], p_msg[1]['b'].reshape(1, 64)
    w3m, b3m = p_msg[2]['w'], p_msg[2]['b'].reshape(1, 128)
    full = lambda a, b: pl.BlockSpec((a, b), lambda i: (0, 0))
    return pl.pallas_call(
        _k3_body,
        grid=(grid,),
        in_specs=[
            pl.BlockSpec((bm, 128), lambda i: (i, 0)),
            pl.BlockSpec((bm, 1), lambda i: (i, 0)),
            full(1, 128),
            full(64, 64), full(1, 64), full(64, 128), full(1, 128),
            full(64, 64), full(1, 64), full(64, 128), full(1, 128),
        ],
        out_specs=pl.BlockSpec((2, bm, 128), lambda i: (0, i, 0)),
        out_shape=jax.ShapeDtypeStruct((2, E_N, 128), F32),
    )(pre, dis, wd, w2l, b2l, w3l, b3l, w2m, b2m, w3m, b3m)


CH4 = 80
EPT = E_N // NSUB


@functools.cache
def _make_k4():
    return pl.kernel(
        _k4_body,
        out_type=jax.ShapeDtypeStruct((2, NS_N, 128), F32),
        mesh=_sc_mesh(),
        scratch_types=[
            pltpu.VMEM((CH4,), jnp.int32),
            pltpu.VMEM((CH4, 128), F32),
            pltpu.VMEM_SHARED((NS_N, 128), F32),
        ],
    )


def _k4_scatter(dst, edges, zeros):
    return _make_k4()(dst, edges, zeros)


def _k4_body(dst_hbm, edges_hbm, zeros_hbm, out_hbm, idx_v, rows_v, acc):
    c = lax.axis_index("c")
    s = lax.axis_index("s")

    @pl.when(s < NSTRIPE)
    def _zero():
        pltpu.sync_copy(zeros_hbm, acc.at[pl.ds(s * NPT, NPT)])

    plsc.subcore_barrier()

    def chunk(i, _):
        base = s * EPT + i * CH4
        pltpu.sync_copy(dst_hbm.at[pl.ds(base, CH4)], idx_v)
        pltpu.sync_copy(edges_hbm.at[c, pl.ds(base, CH4)], rows_v)
        pltpu.sync_copy(rows_v, acc.at[idx_v], add=True)
        return 0

    lax.fori_loop(0, EPT // CH4, chunk, 0)
    plsc.subcore_barrier()

    @pl.when(s < NSTRIPE)
    def _flush():
        pltpu.sync_copy(acc.at[pl.ds(s * NPT, NPT)],
                        out_hbm.at[c, pl.ds(s * NPT, NPT)])


def _k5_body(xs_ref, sa_ref, ss_ref, w1_ref, wsu_ref, wsh_ref, b1_ref,
             w2_ref, b2_ref, w3_ref, b3_ref, out_ref):
    su = sa_ref[1] / (sa_ref[0] + 1e-16)
    sh = ss_ref[1] / (ss_ref[0] + 1e-16)
    t = (jnp.dot(xs_ref[...], w1_ref[...], preferred_element_type=F32)
         + jnp.dot(su, wsu_ref[...], preferred_element_type=F32)
         + jnp.dot(sh, wsh_ref[...], preferred_element_type=F32)
         + b1_ref[...])
    t = jnp.tanh(t)
    t = jnp.tanh(jnp.dot(t, w2_ref[...], preferred_element_type=F32) + b2_ref[...])
    out_ref[...] = jnp.dot(t, w3_ref[...], preferred_element_type=F32) + b3_ref[...]


def _node_update(xs, sums_a, sums_s, p_upd):
    bm = 1000
    grid = NS_N // bm
    w1 = _pad_rows(p_upd[0]['w'][0:130])
    wsu = p_upd[0]['w'][130:258]
    wsh = p_upd[0]['w'][258:386]
    b1 = p_upd[0]['b'].reshape(1, 64)
    w2, b2 = p_upd[1]['w'], p_upd[1]['b'].reshape(1, 64)
    w3, b3 = p_upd[2]['w'], p_upd[2]['b'].reshape(1, 128)
    full = lambda a, b: pl.BlockSpec((a, b), lambda i: (0, 0))
    return pl.pallas_call(
        _k5_body,
        grid=(grid,),
        in_specs=[
            pl.BlockSpec((bm, 256), lambda i: (i, 0)),
            pl.BlockSpec((2, bm, 128), lambda i: (0, i, 0)),
            pl.BlockSpec((2, bm, 128), lambda i: (0, i, 0)),
            full(256, 64), full(128, 64), full(128, 64), full(1, 64),
            full(64, 64), full(1, 64), full(64, 128), full(1, 128),
        ],
        out_specs=pl.BlockSpec((bm, 128), lambda i: (i, 0)),
        out_shape=jax.ShapeDtypeStruct((NS_N, 128), F32),
    )(xs, sums_a, sums_s, w1, wsu, wsh, b1, w2, b2, w3, b3)


def kernel(h, u, state_pos, action_pos, a2s_edge_index, s2s_edge_index,
           a2s_dis, s2s_dis, params):
    src_a = a2s_edge_index[0].astype(jnp.int32)
    dst_a = a2s_edge_index[1].astype(jnp.int32)
    src_s = s2s_edge_index[0].astype(jnp.int32)
    dst_s = s2s_edge_index[1].astype(jnp.int32)

    wsrc_a, wdst_a, wdis_a, b_a = _split_first_layer(
        params['u2h_logit'], params['u2h_u'])
    wsrc_s, wdst_s, wdis_s, b_s = _split_first_layer(
        params['h2h_logit'], params['h2h_h'])

    xs = jnp.pad(jnp.concatenate([state_pos, h], axis=1), ((0, 0), (0, 126)))
    xa = jnp.pad(jnp.concatenate([action_pos, u], axis=1), ((0, 0), (0, 126)))
    ws = jnp.concatenate(
        [_pad_rows(wdst_a), _pad_rows(wsrc_s), _pad_rows(wdst_s)], axis=1)
    bs = jnp.concatenate([b_a, jnp.zeros((128,), F32), b_s]).reshape(1, 384)
    wa = _pad_rows(wsrc_a)

    q, r, s, p = _node_proj(xs, xa, ws, bs, wa)

    pre_a = _k2_gather(src_a, dst_a, p, q)
    edges_a = _edge_mlp(pre_a, a2s_dis, wdis_a,
                        params['u2h_logit'], params['u2h_u'])
    pre_s = _k2_gather(src_s, dst_s, r, s)
    edges_s = _edge_mlp(pre_s, s2s_dis, wdis_s,
                        params['h2h_logit'], params['h2h_h'])

    zeros = jnp.zeros((NPT, 128), F32)
    sums_a = _k4_scatter(dst_a, edges_a, zeros)
    sums_s = _k4_scatter(dst_s, edges_s, zeros)

    return _node_update(xs, sums_a, sums_s, params['h_upd'])


# speedup vs baseline: 5.9148x; 1.1411x over previous
import functools

import jax
import jax.numpy as jnp
from jax import lax
from jax.experimental import pallas as pl
from jax.experimental.pallas import tpu as pltpu
from jax.experimental.pallas import tpu_sc as plsc

NS_N = 10000
E_N = 160000
F32 = jnp.float32

NC = 2
NSUB = 16
NW = NC * NSUB
CH = 128
NCHUNKS = E_N // CH
NPT = 1000
NSTRIPE = NS_N // NPT


def _pad_rows(w, total=256):
    return jnp.pad(w, ((0, total - w.shape[0]), (0, 0)))


def _split_first_layer(p_logit, p_msg):
    wl, bl = p_logit[0]['w'], p_logit[0]['b']
    wm, bm = p_msg[0]['w'], p_msg[0]['b']

    def side(lo1, hi1, lo2, hi2):
        return jnp.concatenate([
            jnp.concatenate([wl[lo1:hi1], wl[lo2:hi2]], axis=0),
            jnp.concatenate([wm[lo1:hi1], wm[lo2:hi2]], axis=0),
        ], axis=1)

    w_src = side(0, 2, 5, 133)
    w_dst = side(2, 4, 133, 261)
    w_dis = jnp.concatenate([wl[4], wm[4]]).reshape(1, 128)
    b = jnp.concatenate([bl, bm])
    return w_src, w_dst, w_dis, b


def _k1_body(xs_ref, xa_ref, ws_ref, bs_ref, wa_ref, q_ref, r_ref, s_ref, p_ref):
    y = jnp.dot(xs_ref[...], ws_ref[...], preferred_element_type=F32) + bs_ref[...]
    q_ref[...] = y[:, 0:128]
    r_ref[...] = y[:, 128:256]
    s_ref[...] = y[:, 256:384]
    p_ref[...] = jnp.dot(xa_ref[...], wa_ref[...], preferred_element_type=F32)


def _node_proj(xs, xa, ws, bs, wa):
    bm = 1000
    grid = NS_N // bm
    out = jax.ShapeDtypeStruct((NS_N, 128), F32)
    return pl.pallas_call(
        _k1_body,
        grid=(grid,),
        in_specs=[
            pl.BlockSpec((bm, 256), lambda i: (i, 0)),
            pl.BlockSpec((bm, 256), lambda i: (i, 0)),
            pl.BlockSpec((256, 384), lambda i: (0, 0)),
            pl.BlockSpec((1, 384), lambda i: (0, 0)),
            pl.BlockSpec((256, 128), lambda i: (0, 0)),
        ],
        out_specs=[pl.BlockSpec((bm, 128), lambda i: (i, 0))] * 4,
        out_shape=[out, out, out, out],
    )(xs, xa, ws, bs, wa)


@functools.cache
def _sc_mesh():
    return plsc.VectorSubcoreMesh(core_axis_name="c", subcore_axis_name="s")


NPAIR2 = (NCHUNKS // NW) // 2
NREM2 = NCHUNKS - (NCHUNKS // NW) * NW


@functools.cache
def _make_k2():
    return pl.kernel(
        _k2_body,
        out_type=jax.ShapeDtypeStruct((E_N, 128), F32),
        mesh=_sc_mesh(),
        scratch_types=[
            pltpu.VMEM((2, CH), jnp.int32),
            pltpu.VMEM((2, CH), jnp.int32),
            pltpu.VMEM((2, CH, 128), F32),
            pltpu.VMEM((2, CH, 128), F32),
            pltpu.SemaphoreType.DMA,
            pltpu.SemaphoreType.DMA,
            pltpu.SemaphoreType.DMA,
            pltpu.SemaphoreType.DMA,
        ],
    )


def _k2_gather(src, dst, ptab, qtab):
    return _make_k2()(src, dst, ptab, qtab)


def _k2_body(src_hbm, dst_hbm, ptab, qtab, out_hbm,
             idx_s2, idx_d2, rows_s2, rows_d2, sem_s0, sem_d0, sem_s1, sem_d1):
    wid = lax.axis_index("s") * NC + lax.axis_index("c")
    sems = ((sem_s0, sem_d0), (sem_s1, sem_d1))

    def load_idx(ci, b):
        base = (wid + NW * ci) * CH
        pltpu.sync_copy(src_hbm.at[pl.ds(base, CH)], idx_s2.at[b])
        pltpu.sync_copy(dst_hbm.at[pl.ds(base, CH)], idx_d2.at[b])

    def fire(b):
        cs = pltpu.async_copy(ptab.at[idx_s2.at[b]], rows_s2.at[b], sems[b][0])
        cd = pltpu.async_copy(qtab.at[idx_d2.at[b]], rows_d2.at[b], sems[b][1])
        return cs, cd

    def compute_store(ci, b):
        def erow(e, _):
            for j in range(8):
                sl = pl.ds(j * 16, 16)
                rows_s2[b, e, sl] = rows_s2[b, e, sl] + rows_d2[b, e, sl]
            return 0

        lax.fori_loop(0, CH, erow, 0)
        base = (wid + NW * ci) * CH
        pltpu.sync_copy(rows_s2.at[b], out_hbm.at[pl.ds(base, CH)])

    def run_one(ci, b):
        load_idx(ci, b)
        ds_, dd_ = fire(b)
        ds_.wait()
        dd_.wait()
        compute_store(ci, b)

    def pair(j, _):
        i0 = 2 * j
        load_idx(i0, 0)
        a0, a1 = fire(0)
        load_idx(i0 + 1, 1)
        a0.wait()
        a1.wait()
        b0, b1 = fire(1)
        compute_store(i0, 0)
        b0.wait()
        b1.wait()
        compute_store(i0 + 1, 1)
        return 0

    lax.fori_loop(0, NPAIR2, pair, 0)
    run_one(2 * NPAIR2, 0)

    @pl.when(wid < NREM2)
    def _tail():
        run_one(2 * NPAIR2 + 1, 0)


def _k3_body(pre_ref, dis_ref, wd_ref, w2l, b2l, w3l, b3l, w2m, b2m, w3m, b3m,
             out_ref):
    pre = pre_ref[...] + dis_ref[...] * wd_ref[...]
    xl = jnp.tanh(pre[:, 0:64])
    xm = jnp.tanh(pre[:, 64:128])
    tl = jnp.tanh(jnp.dot(xl, w2l[...], preferred_element_type=F32) + b2l[...])
    logit = jnp.dot(tl, w3l[...], preferred_element_type=F32) + b3l[...]
    e = jnp.exp(logit)
    tm = jnp.tanh(jnp.dot(xm, w2m[...], preferred_element_type=F32) + b2m[...])
    msg = jnp.dot(tm, w3m[...], preferred_element_type=F32) + b3m[...]
    out_ref[0] = e
    out_ref[1] = e * msg


def _edge_mlp(pre, dis, wd, p_logit, p_msg):
    bm = 640
    grid = E_N // bm
    w2l, b2l = p_logit[1]['w'], p_logit[1]['b'].reshape(1, 64)
    w3l, b3l = p_logit[2]['w'], p_logit[2]['b'].reshape(1, 128)
    w2m, b2m = p_msg[1]['w'], p_msg[1]['b'].reshape(1, 64)
    w3m, b3m = p_msg[2]['w'], p_msg[2]['b'].reshape(1, 128)
    full = lambda a, b: pl.BlockSpec((a, b), lambda i: (0, 0))
    return pl.pallas_call(
        _k3_body,
        grid=(grid,),
        in_specs=[
            pl.BlockSpec((bm, 128), lambda i: (i, 0)),
            pl.BlockSpec((bm, 1), lambda i: (i, 0)),
            full(1, 128),
            full(64, 64), full(1, 64), full(64, 128), full(1, 128),
            full(64, 64), full(1, 64), full(64, 128), full(1, 128),
        ],
        out_specs=pl.BlockSpec((2, bm, 128), lambda i: (0, i, 0)),
        out_shape=jax.ShapeDtypeStruct((2, E_N, 128), F32),
    )(pre, dis, wd, w2l, b2l, w3l, b3l, w2m, b2m, w3m, b3m)


CH4 = 128
NCH4 = E_N // CH4
NPAIR4 = (NCH4 // NSUB) // 2
NREM4 = NCH4 - (NCH4 // NSUB) * NSUB


@functools.cache
def _make_k4():
    return pl.kernel(
        _k4_body,
        out_type=jax.ShapeDtypeStruct((2, NS_N, 128), F32),
        mesh=_sc_mesh(),
        scratch_types=[
            pltpu.VMEM((2, CH4), jnp.int32),
            pltpu.VMEM((2, CH4, 128), F32),
            pltpu.VMEM_SHARED((NS_N, 128), F32),
            pltpu.SemaphoreType.DMA,
            pltpu.SemaphoreType.DMA,
        ],
    )


def _k4_scatter(dst, edges, zeros):
    return _make_k4()(dst, edges, zeros)


def _k4_body(dst_hbm, edges_hbm, zeros_hbm, out_hbm, idx2, rows2, acc,
             sem0, sem1):
    c = lax.axis_index("c")
    s = lax.axis_index("s")
    sems = (sem0, sem1)

    @pl.when(s < NSTRIPE)
    def _zero():
        pltpu.sync_copy(zeros_hbm, acc.at[pl.ds(s * NPT, NPT)])

    plsc.subcore_barrier()

    def load(ci, b):
        base = (s + NSUB * ci) * CH4
        pltpu.sync_copy(dst_hbm.at[pl.ds(base, CH4)], idx2.at[b])
        return pltpu.async_copy(edges_hbm.at[c, pl.ds(base, CH4)],
                                rows2.at[b], sems[b])

    def scat(b):
        pltpu.sync_copy(rows2.at[b], acc.at[idx2.at[b]], add=True)

    def pair(j, _):
        ra = load(2 * j, 0)
        rb = load(2 * j + 1, 1)
        ra.wait()
        scat(0)
        rb.wait()
        scat(1)
        return 0

    lax.fori_loop(0, NPAIR4, pair, 0)

    @pl.when(s < NREM4)
    def _tail():
        rt = load(2 * NPAIR4, 0)
        rt.wait()
        scat(0)

    plsc.subcore_barrier()

    @pl.when(s < NSTRIPE)
    def _flush():
        pltpu.sync_copy(acc.at[pl.ds(s * NPT, NPT)],
                        out_hbm.at[c, pl.ds(s * NPT, NPT)])


def _k5_body(xs_ref, sa_ref, ss_ref, w1_ref, wsu_ref, wsh_ref, b1_ref,
             w2_ref, b2_ref, w3_ref, b3_ref, out_ref):
    su = sa_ref[1] / (sa_ref[0] + 1e-16)
    sh = ss_ref[1] / (ss_ref[0] + 1e-16)
    t = (jnp.dot(xs_ref[...], w1_ref[...], preferred_element_type=F32)
         + jnp.dot(su, wsu_ref[...], preferred_element_type=F32)
         + jnp.dot(sh, wsh_ref[...], preferred_element_type=F32)
         + b1_ref[...])
    t = jnp.tanh(t)
    t = jnp.tanh(jnp.dot(t, w2_ref[...], preferred_element_type=F32) + b2_ref[...])
    out_ref[...] = jnp.dot(t, w3_ref[...], preferred_element_type=F32) + b3_ref[...]


def _node_update(xs, sums_a, sums_s, p_upd):
    bm = 1000
    grid = NS_N // bm
    w1 = _pad_rows(p_upd[0]['w'][0:130])
    wsu = p_upd[0]['w'][130:258]
    wsh = p_upd[0]['w'][258:386]
    b1 = p_upd[0]['b'].reshape(1, 64)
    w2, b2 = p_upd[1]['w'], p_upd[1]['b'].reshape(1, 64)
    w3, b3 = p_upd[2]['w'], p_upd[2]['b'].reshape(1, 128)
    full = lambda a, b: pl.BlockSpec((a, b), lambda i: (0, 0))
    return pl.pallas_call(
        _k5_body,
        grid=(grid,),
        in_specs=[
            pl.BlockSpec((bm, 256), lambda i: (i, 0)),
            pl.BlockSpec((2, bm, 128), lambda i: (0, i, 0)),
            pl.BlockSpec((2, bm, 128), lambda i: (0, i, 0)),
            full(256, 64), full(128, 64), full(128, 64), full(1, 64),
            full(64, 64), full(1, 64), full(64, 128), full(1, 128),
        ],
        out_specs=pl.BlockSpec((bm, 128), lambda i: (i, 0)),
        out_shape=jax.ShapeDtypeStruct((NS_N, 128), F32),
    )(xs, sums_a, sums_s, w1, wsu, wsh, b1, w2, b2, w3, b3)


def kernel(h, u, state_pos, action_pos, a2s_edge_index, s2s_edge_index,
           a2s_dis, s2s_dis, params):
    src_a = a2s_edge_index[0].astype(jnp.int32)
    dst_a = a2s_edge_index[1].astype(jnp.int32)
    src_s = s2s_edge_index[0].astype(jnp.int32)
    dst_s = s2s_edge_index[1].astype(jnp.int32)

    wsrc_a, wdst_a, wdis_a, b_a = _split_first_layer(
        params['u2h_logit'], params['u2h_u'])
    wsrc_s, wdst_s, wdis_s, b_s = _split_first_layer(
        params['h2h_logit'], params['h2h_h'])

    xs = jnp.pad(jnp.concatenate([state_pos, h], axis=1), ((0, 0), (0, 126)))
    xa = jnp.pad(jnp.concatenate([action_pos, u], axis=1), ((0, 0), (0, 126)))
    ws = jnp.concatenate(
        [_pad_rows(wdst_a), _pad_rows(wsrc_s), _pad_rows(wdst_s)], axis=1)
    bs = jnp.concatenate([b_a, jnp.zeros((128,), F32), b_s]).reshape(1, 384)
    wa = _pad_rows(wsrc_a)

    q, r, s, p = _node_proj(xs, xa, ws, bs, wa)

    pre_a = _k2_gather(src_a, dst_a, p, q)
    edges_a = _edge_mlp(pre_a, a2s_dis, wdis_a,
                        params['u2h_logit'], params['u2h_u'])
    pre_s = _k2_gather(src_s, dst_s, r, s)
    edges_s = _edge_mlp(pre_s, s2s_dis, wdis_s,
                        params['h2h_logit'], params['h2h_h'])

    zeros = jnp.zeros((NPT, 128), F32)
    sums_a = _k4_scatter(dst_a, edges_a, zeros)
    sums_s = _k4_scatter(dst_s, edges_s, zeros)

    return _node_update(xs, sums_a, sums_s, params['h_upd'])
